# baseline (speedup 1.0000x reference)
import functools

import jax
import jax.numpy as jnp
from jax import lax
from jax.experimental import pallas as pl
from jax.experimental.pallas import tpu as pltpu
from jax.experimental.pallas import tpu_sc as plsc

L = 16


def _down_body(x_ref, wd_ref, rmat_ref, feats_ref, logits_ref, *, ne, dl):
    feats = jax.lax.dot_general(
        x_ref[...], wd_ref[...],
        dimension_numbers=(((1,), (1,)), ((), ())),
        preferred_element_type=jnp.float32,
    )
    feats_ref[...] = feats
    logits_ref[...] = jnp.dot(feats, rmat_ref[...],
                              preferred_element_type=jnp.float32)


def _routing_body(logits_hbm, gates_hbm, part_hbm,
                  lg_v, gt_v, p_v, *, ne, n_tok, nw, nc):
    cid = lax.axis_index("c")
    sid = lax.axis_index("s")
    wid = sid * nc + cid
    per_w = n_tok // nw
    nchunk = per_w // L
    base = wid * per_w

    pltpu.sync_copy(logits_hbm.at[:, pl.ds(base, per_w)], lg_v)

    zero = jnp.zeros((L,), jnp.float32)
    one = jnp.ones((L,), jnp.float32)
    ninf = jnp.full((L,), -jnp.inf, jnp.float32)
    big = jnp.full((L,), ne, jnp.int32)

    psum = [zero for _ in range(ne)]
    lsum = [zero for _ in range(ne)]

    for j in range(nchunk):
        sl = pl.ds(j * L, L)
        lgs = [lg_v[e, sl] for e in range(ne)]
        m = functools.reduce(jnp.maximum, lgs)
        ezs = [jnp.exp(v - m) for v in lgs]
        ssum = functools.reduce(jnp.add, ezs)
        probs = [v / ssum for v in ezs]
        m1 = functools.reduce(jnp.maximum, probs)
        i1 = functools.reduce(
            jnp.minimum,
            [jnp.where(probs[e] == m1, jnp.full((L,), e, jnp.int32), big)
             for e in range(ne)])
        pmask = [jnp.where(i1 == e, ninf, probs[e]) for e in range(ne)]
        m2 = functools.reduce(jnp.maximum, pmask)
        i2 = functools.reduce(
            jnp.minimum,
            [jnp.where(pmask[e] == m2, jnp.full((L,), e, jnp.int32), big)
             for e in range(ne)])
        tot = m1 + m2
        w1 = m1 / tot
        w2 = m2 / tot
        for e in range(ne):
            ge = jnp.where(i1 == e, w1, jnp.where(i2 == e, w2, zero))
            gt_v[e, sl] = ge
            psum[e] = psum[e] + probs[e]
            lsum[e] = (lsum[e] + jnp.where(i1 == e, one, zero)
                       + jnp.where(i2 == e, one, zero))

    pltpu.sync_copy(gt_v, gates_hbm.at[:, pl.ds(base, per_w)])

    for e in range(ne):
        p_v[e, :] = psum[e]
        p_v[ne + e, :] = lsum[e]
    pltpu.sync_copy(p_v, part_hbm.at[wid])


def _up_body(feats_ref, gates_ref, wu_ref, part_ref, out_ref, aux_ref,
             *, ne, dl, n_tok, d_model, bn, nw):
    i = pl.program_id(0)
    gates = gates_ref[...]

    acc = jnp.zeros((bn, d_model), jnp.float32)
    for e in range(ne):
        fe = feats_ref[:, e * dl:(e + 1) * dl]
        act = 0.5 * fe * (1.0 + jax.lax.erf(fe * (2.0 ** -0.5)))
        ge = act * gates[:, e:e + 1]
        acc = acc + jnp.dot(ge, wu_ref[e * dl:(e + 1) * dl, :],
                            preferred_element_type=jnp.float32)
    out_ref[...] = acc

    @pl.when(i == 0)
    def _init():
        aux_ref[...] = jnp.zeros_like(aux_ref)

    @pl.when(i == pl.num_programs(0) - 1)
    def _fin():
        p = part_ref[...]
        ps = functools.reduce(
            jnp.add, [p[w * 2 * ne:w * 2 * ne + ne, :] for w in range(nw)])
        ls = functools.reduce(
            jnp.add,
            [p[w * 2 * ne + ne:(w + 1) * 2 * ne, :] for w in range(nw)])
        psv = jnp.sum(ps, axis=1, keepdims=True) / n_tok
        lsv = jnp.sum(ls, axis=1, keepdims=True) / n_tok
        aux_ref[...] = ne * jnp.sum(psv * lsv, axis=(0, 1), keepdims=True)


def kernel(x, w_down, router_w, w_up):
    bsz, t, d_model = x.shape
    ne, dl, _ = w_up.shape
    n_tok = bsz * t
    bn = min(256, n_tok)

    x_flat = x.reshape(n_tok, d_model)
    wu_flat = w_up.reshape(ne * dl, d_model)
    eye = jnp.eye(ne, dtype=jnp.float32)
    rmat = (eye[:, None, :] * router_w[0][None, :, None]).reshape(ne * dl, ne)

    feats, logits = pl.pallas_call(
        functools.partial(_down_body, ne=ne, dl=dl),
        grid=(n_tok // bn,),
        in_specs=[
            pl.BlockSpec((bn, d_model), lambda i: (i, 0)),
            pl.BlockSpec((ne * dl, d_model), lambda i: (0, 0)),
            pl.BlockSpec((ne * dl, ne), lambda i: (0, 0)),
        ],
        out_specs=[
            pl.BlockSpec((bn, ne * dl), lambda i: (i, 0)),
            pl.BlockSpec((bn, ne), lambda i: (i, 0)),
        ],
        out_shape=[
            jax.ShapeDtypeStruct((n_tok, ne * dl), jnp.float32),
            jax.ShapeDtypeStruct((n_tok, ne), jnp.float32),
        ],
    )(x_flat, w_down, rmat)

    info = plsc.get_sparse_core_info()
    nc, ns = info.num_cores, info.num_subcores
    nw = nc * ns
    per_w = n_tok // nw
    mesh = plsc.VectorSubcoreMesh(core_axis_name="c", subcore_axis_name="s")
    gates_t, part = pl.kernel(
        functools.partial(_routing_body, ne=ne, n_tok=n_tok, nw=nw, nc=nc),
        mesh=mesh,
        out_type=[
            jax.ShapeDtypeStruct((ne, n_tok), jnp.float32),
            jax.ShapeDtypeStruct((nw, 2 * ne, L), jnp.float32),
        ],
        scratch_types=[
            pltpu.VMEM((ne, per_w), jnp.float32),
            pltpu.VMEM((ne, per_w), jnp.float32),
            pltpu.VMEM((2 * ne, L), jnp.float32),
        ],
    )(logits.T)
    gates = gates_t.T

    out, aux = pl.pallas_call(
        functools.partial(_up_body, ne=ne, dl=dl, n_tok=n_tok,
                          d_model=d_model, bn=bn, nw=nw),
        grid=(n_tok // bn,),
        in_specs=[
            pl.BlockSpec((bn, ne * dl), lambda i: (i, 0)),
            pl.BlockSpec((bn, ne), lambda i: (i, 0)),
            pl.BlockSpec((ne * dl, d_model), lambda i: (0, 0)),
            pl.BlockSpec((nw * 2 * ne, L), lambda i: (0, 0)),
        ],
        out_specs=[
            pl.BlockSpec((bn, d_model), lambda i: (i, 0)),
            pl.BlockSpec((1, 1), lambda i: (0, 0)),
        ],
        out_shape=[
            jax.ShapeDtypeStruct((n_tok, d_model), jnp.float32),
            jax.ShapeDtypeStruct((1, 1), jnp.float32),
        ],
    )(feats, gates, wu_flat, part.reshape(nw * 2 * ne, L))

    return out.reshape(bsz, t, d_model), aux[0, 0]


# speedup vs baseline: 1.0051x; 1.0051x over previous
import functools

import jax
import jax.numpy as jnp
from jax import lax
from jax.experimental import pallas as pl
from jax.experimental.pallas import tpu as pltpu
from jax.experimental.pallas import tpu_sc as plsc

L = 16


def _down_body(x_ref, wd_ref, rmat_ref, feats_ref, logits_ref, *, ne, dl):
    feats = jax.lax.dot_general(
        x_ref[...], wd_ref[...],
        dimension_numbers=(((1,), (1,)), ((), ())),
        preferred_element_type=jnp.float32,
    )
    feats_ref[...] = feats
    logits = jnp.dot(feats, rmat_ref[...],
                     preferred_element_type=jnp.float32)
    logits_ref[...] = logits.T


def _routing_body(logits_hbm, gates_hbm, part_hbm,
                  lg_v, gt_v, p_v, *, ne, n_tok, nw, nc):
    cid = lax.axis_index("c")
    sid = lax.axis_index("s")
    wid = sid * nc + cid
    per_w = n_tok // nw
    nchunk = per_w // L
    base = wid * per_w

    pltpu.sync_copy(logits_hbm.at[:, pl.ds(base, per_w)], lg_v)

    zero = jnp.zeros((L,), jnp.float32)
    one = jnp.ones((L,), jnp.float32)
    ninf = jnp.full((L,), -jnp.inf, jnp.float32)
    big = jnp.full((L,), ne, jnp.int32)

    psum = [zero for _ in range(ne)]
    lsum = [zero for _ in range(ne)]

    for j in range(nchunk):
        sl = pl.ds(j * L, L)
        lgs = [lg_v[e, sl] for e in range(ne)]
        m = functools.reduce(jnp.maximum, lgs)
        ezs = [jnp.exp(v - m) for v in lgs]
        ssum = functools.reduce(jnp.add, ezs)
        probs = [v / ssum for v in ezs]
        m1 = functools.reduce(jnp.maximum, probs)
        i1 = functools.reduce(
            jnp.minimum,
            [jnp.where(probs[e] == m1, jnp.full((L,), e, jnp.int32), big)
             for e in range(ne)])
        pmask = [jnp.where(i1 == e, ninf, probs[e]) for e in range(ne)]
        m2 = functools.reduce(jnp.maximum, pmask)
        i2 = functools.reduce(
            jnp.minimum,
            [jnp.where(pmask[e] == m2, jnp.full((L,), e, jnp.int32), big)
             for e in range(ne)])
        tot = m1 + m2
        w1 = m1 / tot
        w2 = m2 / tot
        for e in range(ne):
            ge = jnp.where(i1 == e, w1, jnp.where(i2 == e, w2, zero))
            gt_v[e, sl] = ge
            psum[e] = psum[e] + probs[e]
            lsum[e] = (lsum[e] + jnp.where(i1 == e, one, zero)
                       + jnp.where(i2 == e, one, zero))

    pltpu.sync_copy(gt_v, gates_hbm.at[:, pl.ds(base, per_w)])

    for e in range(ne):
        p_v[e, :] = psum[e]
        p_v[ne + e, :] = lsum[e]
    pltpu.sync_copy(p_v, part_hbm.at[wid])


def _up_body(feats_ref, gates_ref, wu_ref, part_ref, out_ref, aux_ref,
             *, ne, dl, n_tok, d_model, bn, nw):
    i = pl.program_id(0)
    gates = gates_ref[...].T

    acc = jnp.zeros((bn, d_model), jnp.float32)
    for e in range(ne):
        fe = feats_ref[:, e * dl:(e + 1) * dl]
        act = 0.5 * fe * (1.0 + jax.lax.erf(fe * (2.0 ** -0.5)))
        ge = act * gates[:, e:e + 1]
        acc = acc + jnp.dot(ge, wu_ref[e * dl:(e + 1) * dl, :],
                            preferred_element_type=jnp.float32)
    out_ref[...] = acc

    @pl.when(i == 0)
    def _init():
        aux_ref[...] = jnp.zeros_like(aux_ref)

    @pl.when(i == pl.num_programs(0) - 1)
    def _fin():
        p = part_ref[...]
        ps = functools.reduce(
            jnp.add, [p[w * 2 * ne:w * 2 * ne + ne, :] for w in range(nw)])
        ls = functools.reduce(
            jnp.add,
            [p[w * 2 * ne + ne:(w + 1) * 2 * ne, :] for w in range(nw)])
        psv = jnp.sum(ps, axis=1, keepdims=True) / n_tok
        lsv = jnp.sum(ls, axis=1, keepdims=True) / n_tok
        aux_ref[...] = ne * jnp.sum(psv * lsv, axis=(0, 1), keepdims=True)


def kernel(x, w_down, router_w, w_up):
    bsz, t, d_model = x.shape
    ne, dl, _ = w_up.shape
    n_tok = bsz * t
    bn = min(256, n_tok)

    x_flat = x.reshape(n_tok, d_model)
    wu_flat = w_up.reshape(ne * dl, d_model)
    eye = jnp.eye(ne, dtype=jnp.float32)
    rmat = (eye[:, None, :] * router_w[0][None, :, None]).reshape(ne * dl, ne)

    feats, logits = pl.pallas_call(
        functools.partial(_down_body, ne=ne, dl=dl),
        grid=(n_tok // bn,),
        in_specs=[
            pl.BlockSpec((bn, d_model), lambda i: (i, 0)),
            pl.BlockSpec((ne * dl, d_model), lambda i: (0, 0)),
            pl.BlockSpec((ne * dl, ne), lambda i: (0, 0)),
        ],
        out_specs=[
            pl.BlockSpec((bn, ne * dl), lambda i: (i, 0)),
            pl.BlockSpec((ne, bn), lambda i: (0, i)),
        ],
        out_shape=[
            jax.ShapeDtypeStruct((n_tok, ne * dl), jnp.float32),
            jax.ShapeDtypeStruct((ne, n_tok), jnp.float32),
        ],
    )(x_flat, w_down, rmat)

    info = plsc.get_sparse_core_info()
    nc, ns = info.num_cores, info.num_subcores
    nw = nc * ns
    per_w = n_tok // nw
    mesh = plsc.VectorSubcoreMesh(core_axis_name="c", subcore_axis_name="s")
    gates_t, part = pl.kernel(
        functools.partial(_routing_body, ne=ne, n_tok=n_tok, nw=nw, nc=nc),
        mesh=mesh,
        out_type=[
            jax.ShapeDtypeStruct((ne, n_tok), jnp.float32),
            jax.ShapeDtypeStruct((nw, 2 * ne, L), jnp.float32),
        ],
        scratch_types=[
            pltpu.VMEM((ne, per_w), jnp.float32),
            pltpu.VMEM((ne, per_w), jnp.float32),
            pltpu.VMEM((2 * ne, L), jnp.float32),
        ],
    )(logits)

    out, aux = pl.pallas_call(
        functools.partial(_up_body, ne=ne, dl=dl, n_tok=n_tok,
                          d_model=d_model, bn=bn, nw=nw),
        grid=(n_tok // bn,),
        in_specs=[
            pl.BlockSpec((bn, ne * dl), lambda i: (i, 0)),
            pl.BlockSpec((ne, bn), lambda i: (0, i)),
            pl.BlockSpec((ne * dl, d_model), lambda i: (0, 0)),
            pl.BlockSpec((nw * 2 * ne, L), lambda i: (0, 0)),
        ],
        out_specs=[
            pl.BlockSpec((bn, d_model), lambda i: (i, 0)),
            pl.BlockSpec((1, 1), lambda i: (0, 0)),
        ],
        out_shape=[
            jax.ShapeDtypeStruct((n_tok, d_model), jnp.float32),
            jax.ShapeDtypeStruct((1, 1), jnp.float32),
        ],
    )(feats, gates_t, wu_flat, part.reshape(nw * 2 * ne, L))

    return out.reshape(bsz, t, d_model), aux[0, 0]
